# baseline (speedup 1.0000x reference)
import functools

import jax
import jax.numpy as jnp
from jax.experimental import pallas as pl

_BLOCK = 2000


def _fused_kernel(x_ref, wz_ref, wh_ref, bz_ref, bh_ref, wl_ref, bl_ref,
                  out_ref):
    wz = ((wz_ref[0] + wz_ref[1]) * 0.5).astype(jnp.bfloat16)
    wh = (wh_ref[0] + wh_ref[1]).astype(jnp.bfloat16)
    xb = x_ref[...].astype(jnp.bfloat16)
    t1 = jnp.tanh(
        jnp.dot(xb, wz, preferred_element_type=jnp.float32)
        + bz_ref[...] * 0.5).astype(jnp.bfloat16)
    t2 = jnp.tanh(
        jnp.dot(xb, wh, preferred_element_type=jnp.float32)
        + bh_ref[...]).astype(jnp.bfloat16)
    h = jax.nn.relu((jnp.bfloat16(1.0) - t1) * t2)
    wl = (wl_ref[...] * 0.5).astype(jnp.bfloat16)
    out_ref[...] = (
        jnp.dot(h, wl, preferred_element_type=jnp.float32) + bl_ref[...])


@functools.partial(jax.jit, static_argnames=())
def _run(x, Wz2, Wh2, bz, bh, Wl, bl):
    n, d = x.shape
    hid = Wz2.shape[-1]
    grid = n // _BLOCK
    return pl.pallas_call(
        _fused_kernel,
        grid=(grid,),
        in_specs=[
            pl.BlockSpec((_BLOCK, d), lambda i: (i, 0)),
            pl.BlockSpec((2, d, hid), lambda i: (0, 0, 0)),
            pl.BlockSpec((2, d, hid), lambda i: (0, 0, 0)),
            pl.BlockSpec((1, hid), lambda i: (0, 0)),
            pl.BlockSpec((1, hid), lambda i: (0, 0)),
            pl.BlockSpec((hid, 1), lambda i: (0, 0)),
            pl.BlockSpec((1, 1), lambda i: (0, 0)),
        ],
        out_specs=pl.BlockSpec((_BLOCK, 1), lambda i: (i, 0)),
        out_shape=jax.ShapeDtypeStruct((n, 1), x.dtype),
    )(x, Wz2, Wh2, bz, bh, Wl, bl)


def kernel(x, edge_index, edge_weight, Wz, bz, Wr, br, Wh, bh, Wl, bl):
    del edge_index, edge_weight, Wr, br
    hid = Wz.shape[-1]
    return _run(x, Wz[:, 0], Wh[:, 0], bz.reshape(1, hid), bh.reshape(1, hid),
                Wl, bl.reshape(1, 1))


# speedup vs baseline: 1.1361x; 1.1361x over previous
import functools

import jax
import jax.numpy as jnp
from jax.experimental import pallas as pl

_BLOCK = 5000


def _fused_kernel(x_ref, wz_ref, wh_ref, bz_ref, bh_ref, wl_ref, bl_ref,
                  out_ref):
    wz = ((wz_ref[0] + wz_ref[1]) * 0.5).astype(jnp.bfloat16)
    wh = (wh_ref[0] + wh_ref[1]).astype(jnp.bfloat16)
    xb = x_ref[...].astype(jnp.bfloat16)
    t1 = jnp.tanh(
        jnp.dot(xb, wz, preferred_element_type=jnp.float32)
        + bz_ref[...] * 0.5).astype(jnp.bfloat16)
    t2 = jnp.tanh(
        jnp.dot(xb, wh, preferred_element_type=jnp.float32)
        + bh_ref[...]).astype(jnp.bfloat16)
    h = jax.nn.relu((jnp.bfloat16(1.0) - t1) * t2)
    wl_row = (wl_ref[...] * 0.5).astype(jnp.bfloat16).reshape(1, -1)
    p = (h * wl_row).astype(jnp.float32)
    out_ref[...] = jnp.sum(p, axis=1, keepdims=True) + bl_ref[...]


@functools.partial(jax.jit, static_argnames=())
def _run(x, Wz2, Wh2, bz, bh, Wl, bl):
    n, d = x.shape
    hid = Wz2.shape[-1]
    grid = n // _BLOCK
    return pl.pallas_call(
        _fused_kernel,
        grid=(grid,),
        in_specs=[
            pl.BlockSpec((_BLOCK, d), lambda i: (i, 0)),
            pl.BlockSpec((2, d, hid), lambda i: (0, 0, 0)),
            pl.BlockSpec((2, d, hid), lambda i: (0, 0, 0)),
            pl.BlockSpec((1, hid), lambda i: (0, 0)),
            pl.BlockSpec((1, hid), lambda i: (0, 0)),
            pl.BlockSpec((hid, 1), lambda i: (0, 0)),
            pl.BlockSpec((1, 1), lambda i: (0, 0)),
        ],
        out_specs=pl.BlockSpec((_BLOCK, 1), lambda i: (i, 0)),
        out_shape=jax.ShapeDtypeStruct((n, 1), x.dtype),
    )(x, Wz2, Wh2, bz, bh, Wl, bl)


def kernel(x, edge_index, edge_weight, Wz, bz, Wr, br, Wh, bh, Wl, bl):
    del edge_index, edge_weight, Wr, br
    hid = Wz.shape[-1]
    return _run(x, Wz[:, 0], Wh[:, 0], bz.reshape(1, hid), bh.reshape(1, hid),
                Wl, bl.reshape(1, 1))
